# Initial kernel scaffold; baseline (speedup 1.0000x reference)
import jax
import jax.numpy as jnp
from jax.experimental import pallas as pl


def kernel(input_seqs, poss_item_seqs):
    raise NotImplementedError("write your pallas kernel here")


# speedup vs baseline: 10.1138x; 10.1138x over previous
import functools

import jax
import jax.numpy as jnp
from jax import lax
from jax.experimental import pallas as pl
from jax.experimental.pallas import tpu as pltpu
from jax.experimental.pallas import tpu_sc as plsc

NUM_ITEMS = 100000
N = NUM_ITEMS + 1
NW = 16
LSUB = 16
SUBW = 392
CHUNK = LSUB * SUBW
NPAD = NW * CHUNK
TOK = 1024 * 200
TOKW = TOK // NW
Q = 1024 * 100
QW = Q // NW
B = 1024
NVB = B // LSUB

_mesh = plsc.VectorSubcoreMesh(
    core_axis_name="c", subcore_axis_name="s", num_cores=1)


@functools.partial(
    pl.kernel,
    out_type=jax.ShapeDtypeStruct((Q,), jnp.float32),
    mesh=_mesh,
    compiler_params=pltpu.CompilerParams(needs_layout_passes=False),
    scratch_types=dict(
        countsT=pltpu.VMEM_SHARED((NPAD,), jnp.int32),
        wtab=pltpu.VMEM_SHARED((NW * B,), jnp.int32),
        recipT=pltpu.VMEM_SHARED((NPAD,), jnp.float32),
        tokbuf=pltpu.VMEM((TOKW,), jnp.int32),
        onesb=pltpu.VMEM((TOKW,), jnp.int32),
        counts_c=pltpu.VMEM((CHUNK,), jnp.int32),
        tie_c=pltpu.VMEM((CHUNK,), jnp.int32),
        recip_c=pltpu.VMEM((CHUNK,), jnp.float32),
        hist2d=pltpu.VMEM((LSUB * B,), jnp.int32),
        tmpB=pltpu.VMEM((B,), jnp.int32),
        psw=pltpu.VMEM((B,), jnp.int32),
        htot=pltpu.VMEM((B,), jnp.int32),
        glb=pltpu.VMEM((B,), jnp.int32),
        qout=pltpu.VMEM((QW,), jnp.float32),
        gsem=pltpu.SemaphoreType.DMA,
    ),
)
def _pop_kernel(tok_hbm, q_hbm, out_hbm, *, countsT, wtab, recipT, tokbuf,
                onesb, counts_c, tie_c, recip_c, hist2d, tmpB, psw, htot,
                glb, qout, gsem):
    w = lax.axis_index("s")
    lane = jnp.arange(LSUB, dtype=jnp.int32)
    zeros16 = jnp.zeros((LSUB,), jnp.int32)
    ones16 = jnp.ones((LSUB,), jnp.int32)
    lane_mul = lane * SUBW
    lane_B = lane * B

    def _fill(ref, n, val):
        def body(v, _):
            ref[pl.ds(v * LSUB, LSUB)] = val
            return 0
        lax.fori_loop(0, n // LSUB, body, 0)

    _fill(hist2d, LSUB * B, zeros16)
    _fill(onesb, TOKW, ones16)
    pltpu.sync_copy(hist2d.at[pl.ds(0, CHUNK)],
                    countsT.at[pl.ds(w * CHUNK, CHUNK)])
    plsc.subcore_barrier()

    pltpu.sync_copy(tok_hbm.at[pl.ds(w * TOKW, TOKW)], tokbuf)
    pltpu.sync_copy(onesb, countsT.at[tokbuf], add=True)
    plsc.subcore_barrier()

    pltpu.sync_copy(countsT.at[pl.ds(w * CHUNK, CHUNK)], counts_c)

    def tie_body(t, mx):
        idxs = lane_mul + t
        c = plsc.load_gather(counts_c, [idxs])
        cl = jnp.minimum(c, B - 1)
        hidx = lane_B + cl
        tie = plsc.load_gather(hist2d, [hidx])
        plsc.store_scatter(tie_c, [idxs], tie)
        plsc.addupdate_scatter(hist2d, [hidx], ones16, mask=c < B)
        return jnp.maximum(mx, jnp.max(c))
    maxc = lax.fori_loop(0, SUBW, tie_body, jnp.int32(0))

    def wsum_body(v, _):
        def lbody(l, s):
            sl = hist2d[pl.ds(l * B + v * LSUB, LSUB)]
            hist2d[pl.ds(l * B + v * LSUB, LSUB)] = s
            return s + sl
        tot = lax.fori_loop(0, LSUB, lbody, zeros16)
        tmpB[pl.ds(v * LSUB, LSUB)] = tot
        return 0
    lax.fori_loop(0, NVB, wsum_body, 0)
    pltpu.sync_copy(tmpB, wtab.at[pl.ds(w * B, B)])
    plsc.subcore_barrier()

    _fill(psw, B, zeros16)
    _fill(htot, B, zeros16)

    def wpre_body(j, _):
        pltpu.sync_copy(wtab.at[pl.ds(j * B, B)], tmpB)
        before = j < w

        def vb(v, __):
            sl = pl.ds(v * LSUB, LSUB)
            row = tmpB[sl]
            htot[sl] = htot[sl] + row
            psw[sl] = psw[sl] + jnp.where(before, row, 0)
            return 0
        lax.fori_loop(0, NVB, vb, 0)
        return 0
    lax.fori_loop(0, NW, wpre_body, 0)

    def g_body(v, carry):
        sl = pl.ds(v * LSUB, LSUB)
        vals = htot[sl]
        glb[sl] = NPAD - (plsc.cumsum(vals) + carry)
        return carry + jnp.sum(vals)
    lax.fori_loop(0, NVB, g_body, jnp.int32(0))

    def rank_body(t, _):
        idxs = lane_mul + t
        c = plsc.load_gather(counts_c, [idxs])
        cl = jnp.minimum(c, B - 1)
        g = plsc.load_gather(glb, [cl])
        tie = plsc.load_gather(tie_c, [idxs])
        ps1 = plsc.load_gather(psw, [cl])
        ps2 = plsc.load_gather(hist2d, [lane_B + cl])
        rank = 1 + g + ps1 + ps2 + tie
        plsc.store_scatter(recip_c, [idxs], 1.0 / rank.astype(jnp.float32))
        return 0
    lax.fori_loop(0, SUBW, rank_body, 0)

    @pl.when(maxc >= B)
    def _heavy_fixup():
        def t_body(t, _):
            idxs = lane_mul + t
            c = plsc.load_gather(counts_c, [idxs])
            nh = jnp.sum((c >= B).astype(jnp.int32))

            @pl.when(nh > 0)
            def _():
                for k in range(LSUB):
                    ck = jnp.sum(jnp.where(lane == k, c, 0))

                    @pl.when(ck >= B)
                    def _():
                        gi = w * CHUNK + k * SUBW + t

                        def outer(sw, acc):
                            pltpu.sync_copy(
                                countsT.at[pl.ds(sw * CHUNK, CHUNK)], tie_c)

                            def inner(v, a):
                                cv = tie_c[pl.ds(v * LSUB, LSUB)]
                                gidx = sw * CHUNK + v * LSUB + lane
                                m_gt = cv > ck
                                m_tie = (cv == ck) & (gidx < gi)
                                return (a + jnp.sum(m_gt.astype(jnp.int32))
                                        + jnp.sum(m_tie.astype(jnp.int32)))
                            return lax.fori_loop(0, SUBW, inner, acc)
                        nge = lax.fori_loop(0, NW, outer, jnp.int32(0))
                        rank = (1 + nge).astype(jnp.float32)
                        pos = k * SUBW + t
                        plsc.store_scatter(recip_c, [lane * 0 + pos],
                                           jnp.full((LSUB,), 1.0,
                                                    jnp.float32) / rank,
                                           mask=lane == 0)
            return 0
        lax.fori_loop(0, SUBW, t_body, 0)

    pltpu.sync_copy(recip_c, recipT.at[pl.ds(w * CHUNK, CHUNK)])
    plsc.subcore_barrier()

    pltpu.sync_copy(q_hbm.at[pl.ds(w * QW, QW)], tokbuf.at[pl.ds(0, QW)])
    pltpu.async_copy(recipT.at[tokbuf.at[pl.ds(0, QW)]], qout, gsem).wait()
    pltpu.sync_copy(qout, out_hbm.at[pl.ds(w * QW, QW)])


@jax.jit
def kernel(input_seqs, poss_item_seqs):
    scores = _pop_kernel(input_seqs.reshape(-1), poss_item_seqs.reshape(-1))
    return scores.reshape(poss_item_seqs.shape)


# speedup vs baseline: 14.4933x; 1.4330x over previous
import functools

import jax
import jax.numpy as jnp
from jax import lax
from jax.experimental import pallas as pl
from jax.experimental.pallas import tpu as pltpu
from jax.experimental.pallas import tpu_sc as plsc

NUM_ITEMS = 100000
N = NUM_ITEMS + 1
NW = 16
LSUB = 16
SUBW = 392
CHUNK = LSUB * SUBW
NPAD = NW * CHUNK
TOK = 1024 * 200
TOKW = TOK // NW
Q = 1024 * 100
QW = Q // NW
B = 1024
NVB = B // LSUB

_mesh = plsc.VectorSubcoreMesh(
    core_axis_name="c", subcore_axis_name="s", num_cores=1)


@functools.partial(
    pl.kernel,
    out_type=jax.ShapeDtypeStruct((Q,), jnp.float32),
    mesh=_mesh,
    compiler_params=pltpu.CompilerParams(needs_layout_passes=False),
    scratch_types=dict(
        countsT=pltpu.VMEM_SHARED((NPAD,), jnp.int32),
        wtab=pltpu.VMEM_SHARED((NW * B,), jnp.int32),
        recipT=pltpu.VMEM_SHARED((NPAD,), jnp.float32),
        tokbuf=pltpu.VMEM((TOKW,), jnp.int32),
        onesb=pltpu.VMEM((TOKW,), jnp.int32),
        counts_c=pltpu.VMEM((CHUNK,), jnp.int32),
        tie_c=pltpu.VMEM((CHUNK,), jnp.int32),
        recip_c=pltpu.VMEM((CHUNK,), jnp.float32),
        hist2d=pltpu.VMEM((LSUB * B,), jnp.int32),
        tmpB=pltpu.VMEM((B,), jnp.int32),
        psw=pltpu.VMEM((B,), jnp.int32),
        htot=pltpu.VMEM((B,), jnp.int32),
        glb=pltpu.VMEM((B,), jnp.int32),
        qidx=pltpu.VMEM((QW,), jnp.int32),
        qout=pltpu.VMEM((QW,), jnp.float32),
        gsem=pltpu.SemaphoreType.DMA,
        tsem=pltpu.SemaphoreType.DMA,
    ),
)
def _pop_kernel(tok_hbm, q_hbm, out_hbm, *, countsT, wtab, recipT, tokbuf,
                onesb, counts_c, tie_c, recip_c, hist2d, tmpB, psw, htot,
                glb, qidx, qout, gsem, tsem):
    w = lax.axis_index("s")
    lane = jnp.arange(LSUB, dtype=jnp.int32)
    zeros16 = jnp.zeros((LSUB,), jnp.int32)
    ones16 = jnp.ones((LSUB,), jnp.int32)
    lane_mul = lane * SUBW
    lane_B = lane * B

    tok_dma = pltpu.async_copy(tok_hbm.at[pl.ds(w * TOKW, TOKW)], tokbuf,
                               tsem)
    q_dma = pltpu.async_copy(q_hbm.at[pl.ds(w * QW, QW)], qidx, gsem)

    def _fill(ref, n, val):
        @plsc.parallel_loop(0, n // LSUB, unroll=8)
        def body(v):
            ref[pl.ds(v * LSUB, LSUB)] = val

    _fill(hist2d, LSUB * B, zeros16)
    _fill(onesb, TOKW, ones16)
    pltpu.sync_copy(hist2d.at[pl.ds(0, CHUNK)],
                    countsT.at[pl.ds(w * CHUNK, CHUNK)])
    plsc.subcore_barrier()

    tok_dma.wait()
    pltpu.sync_copy(onesb, countsT.at[tokbuf], add=True)
    plsc.subcore_barrier()

    pltpu.sync_copy(countsT.at[pl.ds(w * CHUNK, CHUNK)], counts_c)

    def tie_body(t, mx):
        idxs = lane_mul + t
        c = plsc.load_gather(counts_c, [idxs])
        cl = jnp.minimum(c, B - 1)
        hidx = lane_B + cl
        tie = plsc.load_gather(hist2d, [hidx])
        plsc.store_scatter(tie_c, [idxs], tie)
        plsc.addupdate_scatter(hist2d, [hidx], ones16, mask=c < B)
        return jnp.maximum(mx, c)
    maxv = lax.fori_loop(0, SUBW, tie_body, zeros16)
    maxc = jnp.max(maxv)

    @plsc.parallel_loop(0, NVB, unroll=2)
    def wsum_body(v):
        def lbody(l, s):
            sl = hist2d[pl.ds(l * B + v * LSUB, LSUB)]
            hist2d[pl.ds(l * B + v * LSUB, LSUB)] = s
            return s + sl
        tot = lax.fori_loop(0, LSUB, lbody, zeros16)
        tmpB[pl.ds(v * LSUB, LSUB)] = tot
    pltpu.sync_copy(tmpB, wtab.at[pl.ds(w * B, B)])
    plsc.subcore_barrier()

    _fill(psw, B, zeros16)
    _fill(htot, B, zeros16)

    def wpre_body(j, _):
        pltpu.sync_copy(wtab.at[pl.ds(j * B, B)], tmpB)
        before = j < w

        @plsc.parallel_loop(0, NVB, unroll=4)
        def vb(v):
            sl = pl.ds(v * LSUB, LSUB)
            row = tmpB[sl]
            htot[sl] = htot[sl] + row
            psw[sl] = psw[sl] + jnp.where(before, row, 0)
        return 0
    lax.fori_loop(0, NW, wpre_body, 0)

    @plsc.parallel_loop(0, NVB, carry=jnp.int32(0))
    def g_body(v, carry):
        sl = pl.ds(v * LSUB, LSUB)
        vals = htot[sl]
        glb[sl] = (NPAD + 1) - (plsc.cumsum(vals) + carry) + psw[sl]
        return carry + jnp.sum(vals)

    @plsc.parallel_loop(0, SUBW, unroll=4)
    def rank_body(t):
        idxs = lane_mul + t
        c = plsc.load_gather(counts_c, [idxs])
        cl = jnp.minimum(c, B - 1)
        g = plsc.load_gather(glb, [cl])
        tie = plsc.load_gather(tie_c, [idxs])
        ps2 = plsc.load_gather(hist2d, [lane_B + cl])
        rank = g + ps2 + tie
        plsc.store_scatter(recip_c, [idxs], 1.0 / rank.astype(jnp.float32))

    @pl.when(maxc >= B)
    def _heavy_fixup():
        def t_body(t, _):
            idxs = lane_mul + t
            c = plsc.load_gather(counts_c, [idxs])
            nh = jnp.sum((c >= B).astype(jnp.int32))

            @pl.when(nh > 0)
            def _():
                for k in range(LSUB):
                    ck = jnp.sum(jnp.where(lane == k, c, 0))

                    @pl.when(ck >= B)
                    def _():
                        gi = w * CHUNK + k * SUBW + t

                        def outer(sw, acc):
                            pltpu.sync_copy(
                                countsT.at[pl.ds(sw * CHUNK, CHUNK)], tie_c)

                            def inner(v, a):
                                cv = tie_c[pl.ds(v * LSUB, LSUB)]
                                gidx = sw * CHUNK + v * LSUB + lane
                                m_gt = cv > ck
                                m_tie = (cv == ck) & (gidx < gi)
                                return (a + jnp.sum(m_gt.astype(jnp.int32))
                                        + jnp.sum(m_tie.astype(jnp.int32)))
                            return lax.fori_loop(0, SUBW, inner, acc)
                        nge = lax.fori_loop(0, NW, outer, jnp.int32(0))
                        rank = (1 + nge).astype(jnp.float32)
                        pos = k * SUBW + t
                        plsc.store_scatter(recip_c, [lane * 0 + pos],
                                           jnp.full((LSUB,), 1.0,
                                                    jnp.float32) / rank,
                                           mask=lane == 0)
            return 0
        lax.fori_loop(0, SUBW, t_body, 0)

    pltpu.sync_copy(recip_c, recipT.at[pl.ds(w * CHUNK, CHUNK)])
    plsc.subcore_barrier()

    q_dma.wait()
    pltpu.async_copy(recipT.at[qidx], qout, gsem).wait()
    pltpu.sync_copy(qout, out_hbm.at[pl.ds(w * QW, QW)])


@jax.jit
def kernel(input_seqs, poss_item_seqs):
    scores = _pop_kernel(input_seqs.reshape(-1), poss_item_seqs.reshape(-1))
    return scores.reshape(poss_item_seqs.shape)


# speedup vs baseline: 16.3595x; 1.1288x over previous
import functools

import jax
import jax.numpy as jnp
from jax import lax
from jax.experimental import pallas as pl
from jax.experimental.pallas import tpu as pltpu
from jax.experimental.pallas import tpu_sc as plsc

NUM_ITEMS = 100000
N = NUM_ITEMS + 1
NW = 16
LSUB = 16
SUBW = 392
CHUNK = LSUB * SUBW
NPAD = NW * CHUNK
TOK = 1024 * 200
TOKW = TOK // NW
Q = 1024 * 100
QW = Q // NW
B = 1024
NVB = B // LSUB

_mesh = plsc.VectorSubcoreMesh(
    core_axis_name="c", subcore_axis_name="s", num_cores=1)


@functools.partial(
    pl.kernel,
    out_type=jax.ShapeDtypeStruct((Q,), jnp.float32),
    mesh=_mesh,
    compiler_params=pltpu.CompilerParams(needs_layout_passes=False,
                                         skip_device_barrier=True),
    scratch_types=dict(
        countsT=pltpu.VMEM_SHARED((NPAD,), jnp.int32),
        wtab=pltpu.VMEM_SHARED((NW * B,), jnp.int32),
        recipT=pltpu.VMEM_SHARED((NPAD,), jnp.float32),
        tokbuf=pltpu.VMEM((TOKW,), jnp.int32),
        onesb=pltpu.VMEM((TOKW,), jnp.int32),
        counts_c=pltpu.VMEM((CHUNK,), jnp.int32),
        tie_c=pltpu.VMEM((CHUNK,), jnp.int32),
        recip_c=pltpu.VMEM((CHUNK,), jnp.float32),
        hist2d=pltpu.VMEM((LSUB * B,), jnp.int32),
        tmpB=pltpu.VMEM((B,), jnp.int32),
        psw=pltpu.VMEM((B,), jnp.int32),
        htot=pltpu.VMEM((B,), jnp.int32),
        glb=pltpu.VMEM((B,), jnp.int32),
        wall=pltpu.VMEM((NW * B,), jnp.int32),
        qidx=pltpu.VMEM((QW,), jnp.int32),
        qout=pltpu.VMEM((QW,), jnp.float32),
        gsem=pltpu.SemaphoreType.DMA,
        tsem=pltpu.SemaphoreType.DMA,
    ),
)
def _pop_kernel(tok_hbm, q_hbm, out_hbm, *, countsT, wtab, recipT, tokbuf,
                onesb, counts_c, tie_c, recip_c, hist2d, tmpB, psw, htot,
                glb, wall, qidx, qout, gsem, tsem):
    w = lax.axis_index("s")
    lane = jnp.arange(LSUB, dtype=jnp.int32)
    zeros16 = jnp.zeros((LSUB,), jnp.int32)
    ones16 = jnp.ones((LSUB,), jnp.int32)
    lane_mul = lane * SUBW
    lane_B = lane * B

    tok_dma = pltpu.async_copy(tok_hbm.at[pl.ds(w * TOKW, TOKW)], tokbuf,
                               tsem)
    q_dma = pltpu.async_copy(q_hbm.at[pl.ds(w * QW, QW)], qidx, gsem)

    def _fill(ref, n, val):
        @plsc.parallel_loop(0, n // LSUB, unroll=8)
        def body(v):
            ref[pl.ds(v * LSUB, LSUB)] = val

    _fill(hist2d, LSUB * B, zeros16)
    _fill(onesb, TOKW, ones16)
    pltpu.sync_copy(hist2d.at[pl.ds(0, CHUNK)],
                    countsT.at[pl.ds(w * CHUNK, CHUNK)])
    plsc.subcore_barrier()

    tok_dma.wait()
    pltpu.sync_copy(onesb, countsT.at[tokbuf], add=True)
    plsc.subcore_barrier()

    pltpu.sync_copy(countsT.at[pl.ds(w * CHUNK, CHUNK)], counts_c)

    c0 = plsc.load_gather(counts_c, [lane_mul])

    def tie_body(t, carry):
        c, mx = carry
        c_next = plsc.load_gather(
            counts_c, [lane_mul + jnp.minimum(t + 1, SUBW - 1)])
        cl = jnp.minimum(c, B - 1)
        hidx = lane_B + cl
        tie = plsc.load_gather(hist2d, [hidx])
        plsc.store_scatter(tie_c, [lane_mul + t], tie)
        plsc.addupdate_scatter(hist2d, [hidx], ones16, mask=c < B)
        return c_next, jnp.maximum(mx, c)
    _, maxv = lax.fori_loop(0, SUBW, tie_body, (c0, zeros16))
    maxc = jnp.max(maxv)

    @plsc.parallel_loop(0, NVB, unroll=2)
    def wsum_body(v):
        s = zeros16
        for l in range(LSUB):
            sl = hist2d[pl.ds(l * B + v * LSUB, LSUB)]
            hist2d[pl.ds(l * B + v * LSUB, LSUB)] = s
            s = s + sl
        tmpB[pl.ds(v * LSUB, LSUB)] = s
    pltpu.sync_copy(tmpB, wtab.at[pl.ds(w * B, B)])
    plsc.subcore_barrier()

    pltpu.sync_copy(wtab, wall)

    @plsc.parallel_loop(0, NVB, unroll=2)
    def wpre_body(v):
        sl = pl.ds(v * LSUB, LSUB)
        ht = zeros16
        ps = zeros16
        for j in range(NW):
            row = wall[pl.ds(j * B + v * LSUB, LSUB)]
            ht = ht + row
            ps = ps + jnp.where(j < w, row, 0)
        htot[sl] = ht
        psw[sl] = ps

    @plsc.parallel_loop(0, NVB, carry=jnp.int32(0))
    def g_body(v, carry):
        sl = pl.ds(v * LSUB, LSUB)
        vals = htot[sl]
        glb[sl] = (NPAD + 1) - (plsc.cumsum(vals) + carry) + psw[sl]
        return carry + jnp.sum(vals)

    @plsc.parallel_loop(0, SUBW, unroll=4)
    def rank_body(t):
        idxs = lane_mul + t
        c = plsc.load_gather(counts_c, [idxs])
        cl = jnp.minimum(c, B - 1)
        g = plsc.load_gather(glb, [cl])
        tie = plsc.load_gather(tie_c, [idxs])
        ps2 = plsc.load_gather(hist2d, [lane_B + cl])
        rank = g + ps2 + tie
        plsc.store_scatter(recip_c, [idxs], 1.0 / rank.astype(jnp.float32))

    @pl.when(maxc >= B)
    def _heavy_fixup():
        def t_body(t, _):
            idxs = lane_mul + t
            c = plsc.load_gather(counts_c, [idxs])
            nh = jnp.sum((c >= B).astype(jnp.int32))

            @pl.when(nh > 0)
            def _():
                for k in range(LSUB):
                    ck = jnp.sum(jnp.where(lane == k, c, 0))

                    @pl.when(ck >= B)
                    def _():
                        gi = w * CHUNK + k * SUBW + t

                        def outer(sw, acc):
                            pltpu.sync_copy(
                                countsT.at[pl.ds(sw * CHUNK, CHUNK)], tie_c)

                            def inner(v, a):
                                cv = tie_c[pl.ds(v * LSUB, LSUB)]
                                gidx = sw * CHUNK + v * LSUB + lane
                                m_gt = cv > ck
                                m_tie = (cv == ck) & (gidx < gi)
                                return (a + jnp.sum(m_gt.astype(jnp.int32))
                                        + jnp.sum(m_tie.astype(jnp.int32)))
                            return lax.fori_loop(0, SUBW, inner, acc)
                        nge = lax.fori_loop(0, NW, outer, jnp.int32(0))
                        rank = (1 + nge).astype(jnp.float32)
                        pos = k * SUBW + t
                        plsc.store_scatter(recip_c, [lane * 0 + pos],
                                           jnp.full((LSUB,), 1.0,
                                                    jnp.float32) / rank,
                                           mask=lane == 0)
            return 0
        lax.fori_loop(0, SUBW, t_body, 0)

    pltpu.sync_copy(recip_c, recipT.at[pl.ds(w * CHUNK, CHUNK)])
    plsc.subcore_barrier()

    q_dma.wait()
    pltpu.async_copy(recipT.at[qidx], qout, gsem).wait()
    pltpu.sync_copy(qout, out_hbm.at[pl.ds(w * QW, QW)])


@jax.jit
def kernel(input_seqs, poss_item_seqs):
    scores = _pop_kernel(input_seqs.reshape(-1), poss_item_seqs.reshape(-1))
    return scores.reshape(poss_item_seqs.shape)
